# baseline (speedup 1.0000x reference)
import functools
import math

import jax
import jax.numpy as jnp
from jax.experimental import pallas as pl
from jax.experimental.pallas import tpu as pltpu

_N = 40
_E = 1024
_BF = jnp.bfloat16
_F32 = jnp.float32
_PAD = 16


def _copy(src, dst, sem):
    c = pltpu.make_async_copy(src, dst, sem)
    c.start()
    c.wait()


def _dott(a, b):
    return jax.lax.dot_general(a, b, (((1,), (1,)), ((), ())),
                               preferred_element_type=_F32)


def _conv_block_call(x, ws, bs, th, crw=None, crb=None):
    n_, H, Wp, Cin = x.shape
    W = Wp - 2 * _PAD
    L = len(ws)
    chans = [Cin] + [w.shape[-1] for w in ws]
    nt = H // th
    th2, W2 = th // 2, W // 2
    Wp2 = W2 + 2 * _PAD
    Co = chans[-1]
    rows0 = (th if nt > 1 else H) + 2 * L

    def body(*refs):
        x_hbm = refs[0]
        wrefs = refs[1:1 + L]
        brefs = refs[1 + L:1 + 2 * L]
        k = 1 + 2 * L
        if crw is not None:
            crw_ref, crb_ref = refs[k], refs[k + 1]
            k += 2
        out_ref = refs[k]
        A = refs[k + 1:k + 2 + L]
        sem = refs[k + 2 + L]

        n = pl.program_id(0)
        i = pl.program_id(1)
        A0 = A[0]

        if nt == 1:
            A0[0:L] = jnp.zeros((L, Wp, Cin), _BF)
            A0[L + H:] = jnp.zeros((L, Wp, Cin), _BF)
            _copy(x_hbm.at[n], A0.at[pl.ds(L, H)], sem)
        else:
            @pl.when(i == 0)
            def _():
                A0[0:L] = jnp.zeros((L, Wp, Cin), _BF)
                _copy(x_hbm.at[n, pl.ds(0, th + L)], A0.at[pl.ds(L, th + L)], sem)

            @pl.when(jnp.logical_and(i > 0, i < nt - 1))
            def _():
                _copy(x_hbm.at[n, pl.ds(i * th - L, rows0)], A0, sem)

            @pl.when(i == nt - 1)
            def _():
                _copy(x_hbm.at[n, pl.ds(i * th - L, th + L)], A0.at[pl.ds(0, th + L)], sem)
                A0[th + L:] = jnp.zeros((L, Wp, Cin), _BF)

        src = A0
        rows_in = rows0
        for l in range(L):
            dst = A[l + 1]
            Ci, Cl = chans[l], chans[l + 1]
            rows_out = rows_in - 2
            w_ref, b_ref = wrefs[l], brefs[l]
            dst[:, 0:_PAD, :] = jnp.zeros((rows_out, _PAD, Cl), _BF)
            dst[:, W + _PAD:, :] = jnp.zeros((rows_out, _PAD, Cl), _BF)
            rc = max(1, min(rows_out, max(256, 131072 // Cl) // W))
            for r0 in range(0, rows_out, rc):
                cc = min(rc, rows_out - r0)
                acc = None
                for dy in range(3):
                    for dx in range(3):
                        a = src[r0 + dy:r0 + dy + cc,
                                _PAD - 1 + dx:_PAD - 1 + dx + W, :]
                        lhs = a.reshape(cc * W, Ci)
                        d = jnp.dot(lhs, w_ref[dy, dx],
                                    preferred_element_type=_F32)
                        acc = d if acc is None else acc + d
                z = jnp.maximum(acc + b_ref[...], 0.0)
                dst[r0:r0 + cc, _PAD:_PAD + W, :] = z.reshape(cc, W, Cl).astype(_BF)
            hb = L - 1 - l
            if hb > 0:
                zrow = jnp.zeros((hb, Wp, Cl), _BF)
                if nt == 1:
                    dst[0:hb] = zrow
                    dst[rows_out - hb:] = zrow
                else:
                    @pl.when(i == 0)
                    def _():
                        dst[0:hb] = zrow

                    @pl.when(i == nt - 1)
                    def _():
                        dst[rows_out - hb:] = zrow
            src = dst
            rows_in = rows_out

        trows = th if nt > 1 else H
        pr = min(trows, 8) if crw is None else trows
        for r0 in range(0, trows, pr):
            t = src[r0:r0 + pr, _PAD:_PAD + W, :]
            hp = jnp.max(t.reshape(pr // 2, 2, W, Co), axis=1)
            p = jnp.max(hp.reshape(pr // 2, W2, 2, Co), axis=2)
            if crw is None:
                out_ref[0, r0 // 2:r0 // 2 + pr // 2, _PAD:_PAD + W2, :] = p
            else:
                flat = p.reshape((pr // 2) * W2, Co)
                zc = jnp.dot(flat, crw_ref[...], preferred_element_type=_F32)
                zc = jnp.maximum(zc + crb_ref[...], 0.0)
                out_ref[0] = jnp.transpose(zc).astype(_BF)
        if crw is None:
            out_ref[0, :, 0:_PAD, :] = jnp.zeros((th2, _PAD, Co), _BF)
            out_ref[0, :, W2 + _PAD:, :] = jnp.zeros((th2, _PAD, Co), _BF)

    in_specs = [pl.BlockSpec(memory_space=pl.ANY)]
    operands = [x]
    for w in ws:
        in_specs.append(pl.BlockSpec(w.shape, lambda n, i: (0, 0, 0, 0)))
        operands.append(w)
    for b in bs:
        in_specs.append(pl.BlockSpec(b.shape, lambda n, i: (0, 0)))
        operands.append(b)
    if crw is not None:
        in_specs.append(pl.BlockSpec(crw.shape, lambda n, i: (0, 0)))
        operands.append(crw)
        in_specs.append(pl.BlockSpec(crb.shape, lambda n, i: (0, 0)))
        operands.append(crb)

    if crw is None:
        out_shape = jax.ShapeDtypeStruct((n_, H // 2, Wp2, Co), _BF)
        out_spec = pl.BlockSpec((1, th2, Wp2, Co), lambda n, i: (n, i, 0, 0))
    else:
        out_shape = jax.ShapeDtypeStruct((n_, 128, 64), _BF)
        out_spec = pl.BlockSpec((1, 128, 64), lambda n, i: (n, 0, 0))

    scratch = [pltpu.VMEM((rows0, Wp, Cin), _BF)]
    ri = rows0
    for l in range(L):
        ri -= 2
        scratch.append(pltpu.VMEM((ri, Wp, chans[l + 1]), _BF))
    scratch.append(pltpu.SemaphoreType.DMA)

    return pl.pallas_call(
        body,
        grid=(n_, nt),
        in_specs=in_specs,
        out_specs=out_spec,
        out_shape=out_shape,
        scratch_shapes=scratch,
        compiler_params=pltpu.CompilerParams(
            dimension_semantics=("parallel", "arbitrary")),
    )(*operands)


def _lnorm(v, w, b):
    m = jnp.mean(v, axis=-1, keepdims=True)
    d = v - m
    var = jnp.mean(d * d, axis=-1, keepdims=True)
    return d * jax.lax.rsqrt(var + 1e-5) * w + b


def _head_body(x_ref, d1w_ref, d1b_ref, n1w_ref, n1b_ref, pos_ref,
               ipw_ref, ipb_ref, opw_ref, opb_ref, ln1w_ref, ln1b_ref,
               fp1w_ref, fp1b_ref, fp2w_ref, fp2b_ref, ln2w_ref, ln2b_ref,
               dp2w_ref, dp2b_ref, dp3w_ref, dp3b_ref, mask_ref,
               out_ref, o_sc):
    x = _dott(x_ref[...], d1w_ref[...]) + d1b_ref[...]
    x = jnp.maximum(_lnorm(x, n1w_ref[...], n1b_ref[...]), 0.0)
    x = x + pos_ref[...]

    qkv = _dott(x.astype(_BF), ipw_ref[...]) + ipb_ref[...]
    scale = 1.0 / math.sqrt(_E // 8)
    for h in range(8):
        sl = slice(h * 128, (h + 1) * 128)
        Qh = qkv[:, sl]
        Kh = qkv[:, 1024 + h * 128:1024 + (h + 1) * 128]
        Vh = qkv[:, 2048 + h * 128:2048 + (h + 1) * 128]
        G = _dott(Qh, Kh) * scale + mask_ref[...]
        G = G - jnp.max(G, axis=-1, keepdims=True)
        ex = jnp.exp(G)
        Aw = ex / jnp.sum(ex, axis=-1, keepdims=True)
        o_sc[:, sl] = jnp.dot(Aw, Vh, preferred_element_type=_F32)

    attn = _dott(o_sc[...].astype(_BF), opw_ref[...]) + opb_ref[...]
    y = _lnorm(x + attn, ln1w_ref[...], ln1b_ref[...])
    t = _dott(y, fp1w_ref[...]) + fp1b_ref[...]
    g = 0.5 * t * (1.0 + jax.lax.erf(t * (1.0 / math.sqrt(2.0))))
    p = _dott(g, fp2w_ref[...]) + fp2b_ref[...]
    p = _lnorm(p, ln2w_ref[...], ln2b_ref[...])
    p = _dott(p, dp2w_ref[...]) + dp2b_ref[...]
    p = _dott(p, dp3w_ref[...]) + dp3b_ref[...]
    out_ref[...] = 1.0 / (1.0 + jnp.exp(-p))


def kernel(frames, vgg_w, vgg_b, cr_w, cr_b, d1_w, d1_b, n1_w, n1_b, pos_emb,
           ipw, ipb, opw, opb, ln1_w, ln1_b, fp1_w, fp1_b, fp2_w, fp2_b,
           ln2_w, ln2_b, dp2_w, dp2_b, dp3_w, dp3_b):
    b, s = frames.shape[:2]
    x = frames.reshape(b * s, *frames.shape[2:]).astype(_BF)
    x = x.transpose(0, 2, 3, 1)
    x = jnp.pad(x, ((0, 0), (0, 0), (_PAD, _PAD), (0, 0)))

    wsb = [w.transpose(2, 3, 1, 0).astype(_BF) for w in vgg_w]
    bsb = [bb.reshape(1, -1) for bb in vgg_b]
    crw = cr_w[:, :, 0, 0].T.astype(_BF)
    crb = cr_b.reshape(1, -1)

    x = _conv_block_call(x, wsb[0:2], bsb[0:2], th=32)
    x = _conv_block_call(x, wsb[2:4], bsb[2:4], th=32)
    x = _conv_block_call(x, wsb[4:7], bsb[4:7], th=32)
    x = _conv_block_call(x, wsb[7:10], bsb[7:10], th=32)
    x = _conv_block_call(x, wsb[10:13], bsb[10:13], th=16, crw=crw, crb=crb)
    x2d = x.reshape(_N, 8192)

    posb = jnp.tile(pos_emb, (b, 1))
    r = jnp.arange(_N)
    mask = jnp.where((r[:, None] % s) == (r[None, :] % s), 0.0, -1e30)
    mask = mask.astype(_F32)

    out40 = pl.pallas_call(
        _head_body,
        out_shape=jax.ShapeDtypeStruct((_N, 4), _F32),
        scratch_shapes=[pltpu.VMEM((_N, _E), _F32)],
        compiler_params=pltpu.CompilerParams(),
    )(x2d, d1_w.astype(_BF), d1_b.reshape(1, -1), n1_w.reshape(1, -1),
      n1_b.reshape(1, -1), posb, ipw.astype(_BF), ipb.reshape(1, -1),
      opw.astype(_BF), opb.reshape(1, -1), ln1_w.reshape(1, -1),
      ln1_b.reshape(1, -1), fp1_w, fp1_b.reshape(1, -1), fp2_w,
      fp2_b.reshape(1, -1), ln2_w.reshape(1, -1), ln2_b.reshape(1, -1),
      dp2_w, dp2_b.reshape(1, -1), dp3_w, dp3_b.reshape(1, -1), mask)

    return out40.reshape(b, s, 4)


# speedup vs baseline: 1.4381x; 1.4381x over previous
import functools
import math

import jax
import jax.numpy as jnp
from jax.experimental import pallas as pl
from jax.experimental.pallas import tpu as pltpu

_N = 40
_E = 1024
_BF = jnp.bfloat16
_F32 = jnp.float32
_PAD = 16


def _copy(src, dst, sem):
    c = pltpu.make_async_copy(src, dst, sem)
    c.start()
    c.wait()


def _dott(a, b):
    return jax.lax.dot_general(a, b, (((1,), (1,)), ((), ())),
                               preferred_element_type=_F32)


def _conv_block_call(x, ws, bs, th, crw=None, crb=None):
    n_, H, Wp, Cin = x.shape
    W = Wp - 2 * _PAD
    L = len(ws)
    chans = [Cin] + [w.shape[-1] for w in ws]
    nt = H // th
    th2, W2 = th // 2, W // 2
    Wp2 = W2 + 2 * _PAD
    Co = chans[-1]
    rows0 = (th if nt > 1 else H) + 2 * L

    def body(*refs):
        x_hbm = refs[0]
        wrefs = refs[1:1 + L]
        brefs = refs[1 + L:1 + 2 * L]
        k = 1 + 2 * L
        if crw is not None:
            crw_ref, crb_ref = refs[k], refs[k + 1]
            k += 2
        out_ref = refs[k]
        A = refs[k + 1:k + 3 + L]
        sem = refs[k + 3 + L]

        n = pl.program_id(0)
        i = pl.program_id(1)
        A0 = A[0]
        A0f = A[1]

        if nt == 1:
            A0[0:L] = jnp.zeros((L, Wp, Cin), _BF)
            A0[L + H:] = jnp.zeros((L, Wp, Cin), _BF)
            _copy(x_hbm.at[n], A0.at[pl.ds(L, H)], sem)
        else:
            @pl.when(i == 0)
            def _():
                A0[0:L] = jnp.zeros((L, Wp, Cin), _BF)
                _copy(x_hbm.at[n, pl.ds(0, th + L)], A0.at[pl.ds(L, th + L)], sem)

            @pl.when(jnp.logical_and(i > 0, i < nt - 1))
            def _():
                _copy(x_hbm.at[n, pl.ds(i * th - L, rows0)], A0, sem)

            @pl.when(i == nt - 1)
            def _():
                _copy(x_hbm.at[n, pl.ds(i * th - L, th + L)], A0.at[pl.ds(0, th + L)], sem)
                A0[th + L:] = jnp.zeros((L, Wp, Cin), _BF)

        A0f[...] = A0[...].astype(_F32)

        src = A0f
        rows_in = rows0
        for l in range(L):
            dst = A[l + 2]
            last = l == L - 1
            sdt = _BF if last else _F32
            Ci, Cl = chans[l], chans[l + 1]
            rows_out = rows_in - 2
            w_ref, b_ref = wrefs[l], brefs[l]
            dst[:, 0:_PAD, :] = jnp.zeros((rows_out, _PAD, Cl), sdt)
            dst[:, W + _PAD:, :] = jnp.zeros((rows_out, _PAD, Cl), sdt)
            rc = max(1, min(rows_out, max(256, 131072 // Cl) // W))
            for r0 in range(0, rows_out, rc):
                cc = min(rc, rows_out - r0)
                acc = None
                for dy in range(3):
                    for dx in range(3):
                        a = src[r0 + dy:r0 + dy + cc,
                                _PAD - 1 + dx:_PAD - 1 + dx + W, :]
                        lhs = a.reshape(cc * W, Ci).astype(_BF)
                        d = jnp.dot(lhs, w_ref[dy, dx],
                                    preferred_element_type=_F32)
                        acc = d if acc is None else acc + d
                z = jnp.maximum(acc + b_ref[...], 0.0)
                dst[r0:r0 + cc, _PAD:_PAD + W, :] = z.reshape(cc, W, Cl).astype(sdt)
            hb = L - 1 - l
            if hb > 0:
                zrow = jnp.zeros((hb, Wp, Cl), sdt)
                if nt == 1:
                    dst[0:hb] = zrow
                    dst[rows_out - hb:] = zrow
                else:
                    @pl.when(i == 0)
                    def _():
                        dst[0:hb] = zrow

                    @pl.when(i == nt - 1)
                    def _():
                        dst[rows_out - hb:] = zrow
            src = dst
            rows_in = rows_out

        trows = th if nt > 1 else H
        pr = min(trows, 8) if crw is None else trows
        for r0 in range(0, trows, pr):
            t = src[r0:r0 + pr, _PAD:_PAD + W, :]
            hp = jnp.max(t.reshape(pr // 2, 2, W, Co), axis=1)
            wi = pltpu.bitcast(hp, jnp.int32)
            lo = jnp.bitwise_and(wi, jnp.int32(0xFFFF))
            hi = jax.lax.shift_right_logical(wi, jnp.int32(16))
            m = jnp.maximum(lo, hi)
            p = pltpu.bitcast(jax.lax.shift_left(m, jnp.int32(16)), _F32)
            if crw is None:
                out_ref[0, r0 // 2:r0 // 2 + pr // 2, _PAD:_PAD + W2, :] = p.astype(_BF)
            else:
                flat = p.reshape((pr // 2) * W2, Co).astype(_BF)
                zc = jnp.dot(flat, crw_ref[...], preferred_element_type=_F32)
                zc = jnp.maximum(zc + crb_ref[...], 0.0)
                out_ref[0] = jnp.transpose(zc).astype(_BF)
        if crw is None:
            out_ref[0, :, 0:_PAD, :] = jnp.zeros((th2, _PAD, Co), _BF)
            out_ref[0, :, W2 + _PAD:, :] = jnp.zeros((th2, _PAD, Co), _BF)

    in_specs = [pl.BlockSpec(memory_space=pl.ANY)]
    operands = [x]
    for w in ws:
        in_specs.append(pl.BlockSpec(w.shape, lambda n, i: (0, 0, 0, 0)))
        operands.append(w)
    for b in bs:
        in_specs.append(pl.BlockSpec(b.shape, lambda n, i: (0, 0)))
        operands.append(b)
    if crw is not None:
        in_specs.append(pl.BlockSpec(crw.shape, lambda n, i: (0, 0)))
        operands.append(crw)
        in_specs.append(pl.BlockSpec(crb.shape, lambda n, i: (0, 0)))
        operands.append(crb)

    if crw is None:
        out_shape = jax.ShapeDtypeStruct((n_, H // 2, Wp2, Co), _BF)
        out_spec = pl.BlockSpec((1, th2, Wp2, Co), lambda n, i: (n, i, 0, 0))
    else:
        out_shape = jax.ShapeDtypeStruct((n_, 128, 64), _BF)
        out_spec = pl.BlockSpec((1, 128, 64), lambda n, i: (n, 0, 0))

    scratch = [pltpu.VMEM((rows0, Wp, Cin), _BF),
               pltpu.VMEM((rows0, Wp, Cin), _F32)]
    ri = rows0
    for l in range(L):
        ri -= 2
        scratch.append(pltpu.VMEM((ri, Wp, chans[l + 1]),
                                  _BF if l == L - 1 else _F32))
    scratch.append(pltpu.SemaphoreType.DMA)

    return pl.pallas_call(
        body,
        grid=(n_, nt),
        in_specs=in_specs,
        out_specs=out_spec,
        out_shape=out_shape,
        scratch_shapes=scratch,
        compiler_params=pltpu.CompilerParams(
            dimension_semantics=("parallel", "arbitrary")),
    )(*operands)


def _lnorm(v, w, b):
    m = jnp.mean(v, axis=-1, keepdims=True)
    d = v - m
    var = jnp.mean(d * d, axis=-1, keepdims=True)
    return d * jax.lax.rsqrt(var + 1e-5) * w + b


def _head_body(x_ref, d1w_ref, d1b_ref, n1w_ref, n1b_ref, pos_ref,
               ipw_ref, ipb_ref, opw_ref, opb_ref, ln1w_ref, ln1b_ref,
               fp1w_ref, fp1b_ref, fp2w_ref, fp2b_ref, ln2w_ref, ln2b_ref,
               dp2w_ref, dp2b_ref, dp3w_ref, dp3b_ref, mask_ref,
               out_ref, o_sc):
    x = _dott(x_ref[...], d1w_ref[...]) + d1b_ref[...]
    x = jnp.maximum(_lnorm(x, n1w_ref[...], n1b_ref[...]), 0.0)
    x = x + pos_ref[...]

    qkv = _dott(x.astype(_BF), ipw_ref[...]) + ipb_ref[...]
    scale = 1.0 / math.sqrt(_E // 8)
    for h in range(8):
        sl = slice(h * 128, (h + 1) * 128)
        Qh = qkv[:, sl]
        Kh = qkv[:, 1024 + h * 128:1024 + (h + 1) * 128]
        Vh = qkv[:, 2048 + h * 128:2048 + (h + 1) * 128]
        G = _dott(Qh, Kh) * scale + mask_ref[...]
        G = G - jnp.max(G, axis=-1, keepdims=True)
        ex = jnp.exp(G)
        Aw = ex / jnp.sum(ex, axis=-1, keepdims=True)
        o_sc[:, sl] = jnp.dot(Aw, Vh, preferred_element_type=_F32)

    attn = _dott(o_sc[...].astype(_BF), opw_ref[...]) + opb_ref[...]
    y = _lnorm(x + attn, ln1w_ref[...], ln1b_ref[...])
    t = _dott(y, fp1w_ref[...]) + fp1b_ref[...]
    g = 0.5 * t * (1.0 + jax.lax.erf(t * (1.0 / math.sqrt(2.0))))
    p = _dott(g, fp2w_ref[...]) + fp2b_ref[...]
    p = _lnorm(p, ln2w_ref[...], ln2b_ref[...])
    p = _dott(p, dp2w_ref[...]) + dp2b_ref[...]
    p = _dott(p, dp3w_ref[...]) + dp3b_ref[...]
    out_ref[...] = 1.0 / (1.0 + jnp.exp(-p))


def kernel(frames, vgg_w, vgg_b, cr_w, cr_b, d1_w, d1_b, n1_w, n1_b, pos_emb,
           ipw, ipb, opw, opb, ln1_w, ln1_b, fp1_w, fp1_b, fp2_w, fp2_b,
           ln2_w, ln2_b, dp2_w, dp2_b, dp3_w, dp3_b):
    b, s = frames.shape[:2]
    x = frames.reshape(b * s, *frames.shape[2:]).astype(_BF)
    x = x.transpose(0, 2, 3, 1)
    x = jnp.pad(x, ((0, 0), (0, 0), (_PAD, _PAD), (0, 0)))

    wsb = [w.astype(_BF).transpose(2, 3, 1, 0) for w in vgg_w]
    bsb = [bb.reshape(1, -1) for bb in vgg_b]
    crw = cr_w[:, :, 0, 0].T.astype(_BF)
    crb = cr_b.reshape(1, -1)

    x = _conv_block_call(x, wsb[0:2], bsb[0:2], th=32)
    x = _conv_block_call(x, wsb[2:4], bsb[2:4], th=32)
    x = _conv_block_call(x, wsb[4:7], bsb[4:7], th=32)
    x = _conv_block_call(x, wsb[7:10], bsb[7:10], th=32)
    x = _conv_block_call(x, wsb[10:13], bsb[10:13], th=16, crw=crw, crb=crb)
    x2d = x.reshape(_N, 8192)

    posb = jnp.tile(pos_emb, (b, 1))
    r = jnp.arange(_N)
    mask = jnp.where((r[:, None] % s) == (r[None, :] % s), 0.0, -1e30)
    mask = mask.astype(_F32)

    out40 = pl.pallas_call(
        _head_body,
        out_shape=jax.ShapeDtypeStruct((_N, 4), _F32),
        scratch_shapes=[pltpu.VMEM((_N, _E), _F32)],
        compiler_params=pltpu.CompilerParams(),
    )(x2d, d1_w.astype(_BF), d1_b.reshape(1, -1), n1_w.reshape(1, -1),
      n1_b.reshape(1, -1), posb, ipw.astype(_BF), ipb.reshape(1, -1),
      opw.astype(_BF), opb.reshape(1, -1), ln1_w.reshape(1, -1),
      ln1_b.reshape(1, -1), fp1_w, fp1_b.reshape(1, -1), fp2_w,
      fp2_b.reshape(1, -1), ln2_w.reshape(1, -1), ln2_b.reshape(1, -1),
      dp2_w, dp2_b.reshape(1, -1), dp3_w, dp3_b.reshape(1, -1), mask)

    return out40.reshape(b, s, 4)


# speedup vs baseline: 1.6156x; 1.1235x over previous
import functools
import math

import jax
import jax.numpy as jnp
from jax.experimental import pallas as pl
from jax.experimental.pallas import tpu as pltpu

_N = 40
_E = 1024
_BF = jnp.bfloat16
_F32 = jnp.float32
_PAD = 16


def _copy(src, dst, sem):
    c = pltpu.make_async_copy(src, dst, sem)
    c.start()
    c.wait()


def _dott(a, b):
    return jax.lax.dot_general(a, b, (((1,), (1,)), ((), ())),
                               preferred_element_type=_F32)


def _conv_block_call(x, ws, bs, th, crw=None, crb=None, pad_in=True):
    n_, H, in_w, Cin = x.shape
    W = in_w - 2 * _PAD if pad_in else in_w
    Wp = W + 2 * _PAD
    L = len(ws)
    chans = [Cin] + [w.shape[-1] for w in ws]
    nt = H // th
    th2, W2 = th // 2, W // 2
    Wp2 = W2 + 2 * _PAD
    Co = chans[-1]
    rows0 = (th if nt > 1 else H) + 2 * L

    def body(*refs):
        x_hbm = refs[0]
        wrefs = refs[1:1 + L]
        brefs = refs[1 + L:1 + 2 * L]
        k = 1 + 2 * L
        if crw is not None:
            crw_ref, crb_ref = refs[k], refs[k + 1]
            k += 2
        out_ref = refs[k]
        A = refs[k + 1:k + 3 + L]
        sem = refs[k + 3 + L]

        n = pl.program_id(0)
        i = pl.program_id(1)
        A0 = A[0]
        A0f = A[1]

        def dstq(d0, rows):
            if pad_in:
                return A0.at[pl.ds(d0, rows)]
            return A0.at[pl.ds(d0, rows), pl.ds(_PAD, W)]

        if nt == 1:
            A0[0:L] = jnp.zeros((L, Wp, Cin), _BF)
            A0[L + H:] = jnp.zeros((L, Wp, Cin), _BF)
            _copy(x_hbm.at[n], dstq(L, H), sem)
        else:
            @pl.when(i == 0)
            def _():
                A0[0:L] = jnp.zeros((L, Wp, Cin), _BF)
                _copy(x_hbm.at[n, pl.ds(0, th + L)], dstq(L, th + L), sem)

            @pl.when(jnp.logical_and(i > 0, i < nt - 1))
            def _():
                _copy(x_hbm.at[n, pl.ds(i * th - L, rows0)], dstq(0, rows0), sem)

            @pl.when(i == nt - 1)
            def _():
                _copy(x_hbm.at[n, pl.ds(i * th - L, th + L)], dstq(0, th + L), sem)
                A0[th + L:] = jnp.zeros((L, Wp, Cin), _BF)
        if not pad_in:
            A0[:, 0:_PAD, :] = jnp.zeros((rows0, _PAD, Cin), _BF)
            A0[:, _PAD + W:, :] = jnp.zeros((rows0, _PAD, Cin), _BF)

        A0f[...] = A0[...].astype(_F32)

        src = A0f
        rows_in = rows0
        for l in range(L):
            dst = A[l + 2]
            last = l == L - 1
            sdt = _BF if last else _F32
            Ci, Cl = chans[l], chans[l + 1]
            rows_out = rows_in - 2
            w_ref, b_ref = wrefs[l], brefs[l]
            dst[:, 0:_PAD, :] = jnp.zeros((rows_out, _PAD, Cl), sdt)
            dst[:, W + _PAD:, :] = jnp.zeros((rows_out, _PAD, Cl), sdt)
            rc = max(1, min(rows_out, max(256, 131072 // Cl) // W))
            for r0 in range(0, rows_out, rc):
                cc = min(rc, rows_out - r0)
                acc = None
                for dy in range(3):
                    for dx in range(3):
                        a = src[r0 + dy:r0 + dy + cc,
                                _PAD - 1 + dx:_PAD - 1 + dx + W, :]
                        lhs = a.reshape(cc * W, Ci).astype(_BF)
                        d = jnp.dot(lhs, w_ref[dy, dx],
                                    preferred_element_type=_F32)
                        acc = d if acc is None else acc + d
                z = jnp.maximum(acc + b_ref[...], 0.0)
                dst[r0:r0 + cc, _PAD:_PAD + W, :] = z.reshape(cc, W, Cl).astype(sdt)
            hb = L - 1 - l
            if hb > 0:
                zrow = jnp.zeros((hb, Wp, Cl), sdt)
                if nt == 1:
                    dst[0:hb] = zrow
                    dst[rows_out - hb:] = zrow
                else:
                    @pl.when(i == 0)
                    def _():
                        dst[0:hb] = zrow

                    @pl.when(i == nt - 1)
                    def _():
                        dst[rows_out - hb:] = zrow
            src = dst
            rows_in = rows_out

        trows = th if nt > 1 else H
        pr = min(trows, 8) if crw is None else trows
        for r0 in range(0, trows, pr):
            t = src[r0:r0 + pr, _PAD:_PAD + W, :]
            hp = jnp.max(t.reshape(pr // 2, 2, W, Co), axis=1)
            wi = pltpu.bitcast(hp, jnp.int32)
            lo = jnp.bitwise_and(wi, jnp.int32(0xFFFF))
            hi = jax.lax.shift_right_logical(wi, jnp.int32(16))
            m = jnp.maximum(lo, hi)
            p = pltpu.bitcast(jax.lax.shift_left(m, jnp.int32(16)), _F32)
            if crw is None:
                out_ref[0, r0 // 2:r0 // 2 + pr // 2, _PAD:_PAD + W2, :] = p.astype(_BF)
            else:
                flat = p.reshape((pr // 2) * W2, Co).astype(_BF)
                zc = jnp.dot(flat, crw_ref[...], preferred_element_type=_F32)
                zc = jnp.maximum(zc + crb_ref[...], 0.0)
                out_ref[0] = jnp.transpose(zc).astype(_BF)
        if crw is None:
            out_ref[0, :, 0:_PAD, :] = jnp.zeros((th2, _PAD, Co), _BF)
            out_ref[0, :, W2 + _PAD:, :] = jnp.zeros((th2, _PAD, Co), _BF)

    in_specs = [pl.BlockSpec(memory_space=pl.ANY)]
    operands = [x]
    for w in ws:
        in_specs.append(pl.BlockSpec(w.shape, lambda n, i: (0, 0, 0, 0)))
        operands.append(w)
    for b in bs:
        in_specs.append(pl.BlockSpec(b.shape, lambda n, i: (0, 0)))
        operands.append(b)
    if crw is not None:
        in_specs.append(pl.BlockSpec(crw.shape, lambda n, i: (0, 0)))
        operands.append(crw)
        in_specs.append(pl.BlockSpec(crb.shape, lambda n, i: (0, 0)))
        operands.append(crb)

    if crw is None:
        out_shape = jax.ShapeDtypeStruct((n_, H // 2, Wp2, Co), _BF)
        out_spec = pl.BlockSpec((1, th2, Wp2, Co), lambda n, i: (n, i, 0, 0))
    else:
        out_shape = jax.ShapeDtypeStruct((n_, 128, 64), _BF)
        out_spec = pl.BlockSpec((1, 128, 64), lambda n, i: (n, 0, 0))

    scratch = [pltpu.VMEM((rows0, Wp, Cin), _BF),
               pltpu.VMEM((rows0, Wp, Cin), _F32)]
    ri = rows0
    for l in range(L):
        ri -= 2
        scratch.append(pltpu.VMEM((ri, Wp, chans[l + 1]),
                                  _BF if l == L - 1 else _F32))
    scratch.append(pltpu.SemaphoreType.DMA)

    return pl.pallas_call(
        body,
        grid=(n_, nt),
        in_specs=in_specs,
        out_specs=out_spec,
        out_shape=out_shape,
        scratch_shapes=scratch,
        compiler_params=pltpu.CompilerParams(
            dimension_semantics=("parallel", "arbitrary")),
    )(*operands)


def _lnorm(v, w, b):
    m = jnp.mean(v, axis=-1, keepdims=True)
    d = v - m
    var = jnp.mean(d * d, axis=-1, keepdims=True)
    return d * jax.lax.rsqrt(var + 1e-5) * w + b


def _head_body(x_ref, d1w_ref, d1b_ref, n1w_ref, n1b_ref, pos_ref,
               ipw_ref, ipb_ref, opw_ref, opb_ref, ln1w_ref, ln1b_ref,
               fp1w_ref, fp1b_ref, fp2w_ref, fp2b_ref, ln2w_ref, ln2b_ref,
               dp2w_ref, dp2b_ref, dp3w_ref, dp3b_ref, mask_ref,
               out_ref, o_sc):
    x = _dott(x_ref[...], d1w_ref[...]) + d1b_ref[...]
    x = jnp.maximum(_lnorm(x, n1w_ref[...], n1b_ref[...]), 0.0)
    x = x + pos_ref[...]

    qkv = _dott(x.astype(_BF), ipw_ref[...]) + ipb_ref[...]
    scale = 1.0 / math.sqrt(_E // 8)
    for h in range(8):
        sl = slice(h * 128, (h + 1) * 128)
        Qh = qkv[:, sl]
        Kh = qkv[:, 1024 + h * 128:1024 + (h + 1) * 128]
        Vh = qkv[:, 2048 + h * 128:2048 + (h + 1) * 128]
        G = _dott(Qh, Kh) * scale + mask_ref[...]
        G = G - jnp.max(G, axis=-1, keepdims=True)
        ex = jnp.exp(G)
        Aw = ex / jnp.sum(ex, axis=-1, keepdims=True)
        o_sc[:, sl] = jnp.dot(Aw, Vh, preferred_element_type=_F32)

    attn = _dott(o_sc[...].astype(_BF), opw_ref[...]) + opb_ref[...]
    y = _lnorm(x + attn, ln1w_ref[...], ln1b_ref[...])
    t = _dott(y, fp1w_ref[...]) + fp1b_ref[...]
    g = 0.5 * t * (1.0 + jax.lax.erf(t * (1.0 / math.sqrt(2.0))))
    p = _dott(g, fp2w_ref[...]) + fp2b_ref[...]
    p = _lnorm(p, ln2w_ref[...], ln2b_ref[...])
    p = _dott(p, dp2w_ref[...]) + dp2b_ref[...]
    p = _dott(p, dp3w_ref[...]) + dp3b_ref[...]
    out_ref[...] = 1.0 / (1.0 + jnp.exp(-p))


def kernel(frames, vgg_w, vgg_b, cr_w, cr_b, d1_w, d1_b, n1_w, n1_b, pos_emb,
           ipw, ipb, opw, opb, ln1_w, ln1_b, fp1_w, fp1_b, fp2_w, fp2_b,
           ln2_w, ln2_b, dp2_w, dp2_b, dp3_w, dp3_b):
    b, s = frames.shape[:2]
    x = frames.reshape(b * s, *frames.shape[2:]).astype(_BF)
    x = x.transpose(0, 2, 3, 1)

    wsb = [w.astype(_BF).transpose(2, 3, 1, 0) for w in vgg_w]
    bsb = [bb.reshape(1, -1) for bb in vgg_b]
    crw = cr_w[:, :, 0, 0].T.astype(_BF)
    crb = cr_b.reshape(1, -1)

    x = _conv_block_call(x, wsb[0:2], bsb[0:2], th=64, pad_in=False)
    x = _conv_block_call(x, wsb[2:4], bsb[2:4], th=64)
    x = _conv_block_call(x, wsb[4:7], bsb[4:7], th=64)
    x = _conv_block_call(x, wsb[7:10], bsb[7:10], th=32)
    x = _conv_block_call(x, wsb[10:13], bsb[10:13], th=16, crw=crw, crb=crb)
    x2d = x.reshape(_N, 8192)

    posb = jnp.tile(pos_emb, (b, 1))
    r = jnp.arange(_N)
    mask = jnp.where((r[:, None] % s) == (r[None, :] % s), 0.0, -1e30)
    mask = mask.astype(_F32)

    out40 = pl.pallas_call(
        _head_body,
        out_shape=jax.ShapeDtypeStruct((_N, 4), _F32),
        scratch_shapes=[pltpu.VMEM((_N, _E), _F32)],
        compiler_params=pltpu.CompilerParams(),
    )(x2d, d1_w.astype(_BF), d1_b.reshape(1, -1), n1_w.reshape(1, -1),
      n1_b.reshape(1, -1), posb, ipw.astype(_BF), ipb.reshape(1, -1),
      opw.astype(_BF), opb.reshape(1, -1), ln1_w.reshape(1, -1),
      ln1_b.reshape(1, -1), fp1_w, fp1_b.reshape(1, -1), fp2_w,
      fp2_b.reshape(1, -1), ln2_w.reshape(1, -1), ln2_b.reshape(1, -1),
      dp2_w, dp2_b.reshape(1, -1), dp3_w, dp3_b.reshape(1, -1), mask)

    return out40.reshape(b, s, 4)
